# Initial kernel scaffold; baseline (speedup 1.0000x reference)
import jax
import jax.numpy as jnp
from jax.experimental import pallas as pl


def kernel(x, edge_index, batch, W1a, b1a, W1b, b1b, W1c, b1c, g1, be1, W2a, b2a, W2b, b2b, W2c, b2c, g2, be2, W3a, b3a, W3b, b3b, g3, be3, Wf1, bf1, Wf2, bf2):
    raise NotImplementedError("write your pallas kernel here")


# speedup vs baseline: 12.7136x; 12.7136x over previous
import functools

import jax
import jax.numpy as jnp
from jax import lax
from jax.experimental import pallas as pl
from jax.experimental.pallas import tpu as pltpu
from jax.experimental.pallas import tpu_sc as plsc

N = 10000
E = 320000
D = 128

NC = 2
NS = 16
NW = NC * NS
N_PAD = 10240
K = 80
EPW = 10240
E_PAD = NW * EPW
NCHUNK = EPW // K
RPT = N_PAD // NS
ZR = 16


@functools.lru_cache(maxsize=None)
def _make_segsum(d):
    mesh = plsc.VectorSubcoreMesh(core_axis_name="c", subcore_axis_name="s")

    @functools.partial(
        pl.kernel,
        mesh=mesh,
        compiler_params=pltpu.CompilerParams(use_tc_tiling_on_sc=False),
        out_type=jax.ShapeDtypeStruct((NC * N_PAD, d), jnp.float32),
        scratch_types=[
            pltpu.VMEM((NCHUNK, K), jnp.int32),
            pltpu.VMEM((NCHUNK, K), jnp.int32),
            pltpu.VMEM((K, d), jnp.float32),
            pltpu.VMEM((K, d), jnp.float32),
            pltpu.VMEM((ZR, d), jnp.float32),
            pltpu.VMEM_SHARED((N_PAD, d), jnp.float32),
            pltpu.SemaphoreType.DMA,
            pltpu.SemaphoreType.DMA,
        ],
    )
    def segsum(x_hbm, src_hbm, dst_hbm, out_hbm, src_v, dst_v, buf_a, buf_b,
               zero_v, acc_sh, sem_a, sem_b):
        c = lax.axis_index("c")
        s = lax.axis_index("s")
        gw = c * NS + s

        cp_src = pltpu.async_copy(src_hbm.at[gw], src_v, sem_a)
        cp_dst = pltpu.async_copy(dst_hbm.at[gw], dst_v, sem_b)

        def zrow(i, carry):
            def zcol(j, carry2):
                zero_v[i, pl.ds(j * 16, 16)] = jnp.zeros((16,), jnp.float32)
                return carry2
            return lax.fori_loop(0, d // 16, zcol, carry)
        lax.fori_loop(0, ZR, zrow, 0)

        r0 = s * RPT
        def zacc(i, carry):
            pltpu.sync_copy(zero_v, acc_sh.at[pl.ds(r0 + i * ZR, ZR)])
            return carry
        lax.fori_loop(0, RPT // ZR, zacc, 0)
        cp_src.wait()
        cp_dst.wait()
        plsc.subcore_barrier()

        def gather_start(j, buf, sem):
            return pltpu.async_copy(x_hbm.at[src_v.at[j]], buf, sem)

        def gather_wait(j, buf, sem):
            pltpu.make_async_copy(x_hbm.at[src_v.at[j]], buf, sem).wait()

        def scatter_add(j, buf):
            pltpu.sync_copy(buf, acc_sh.at[dst_v.at[j]], add=True)

        gather_start(0, buf_a, sem_a)

        def body(i, carry):
            j0 = 2 * i
            gather_wait(j0, buf_a, sem_a)
            cp_b = gather_start(j0 + 1, buf_b, sem_b)
            scatter_add(j0, buf_a)

            @pl.when(j0 + 2 < NCHUNK)
            def _():
                gather_start(j0 + 2, buf_a, sem_a)
            cp_b.wait()
            scatter_add(j0 + 1, buf_b)
            return carry
        lax.fori_loop(0, NCHUNK // 2, body, 0)
        plsc.subcore_barrier()

        pltpu.sync_copy(acc_sh.at[pl.ds(r0, RPT)],
                        out_hbm.at[pl.ds(c * N_PAD + r0, RPT)])

    return segsum


def _bn(h, g, b):
    m = jnp.mean(h, axis=0, keepdims=True)
    v = jnp.mean(h * h, axis=0, keepdims=True) - m * m
    return (h - m) * lax.rsqrt(v + 1e-5) * g + b


def _mlp1_body(x, p, W1a, b1a, W1b, b1b, W1c, b1c, g1, be1, out):
    a = x[...] + p[0:N] + p[N_PAD:N_PAD + N]
    h = jnp.maximum(jnp.dot(a, W1a[...], preferred_element_type=jnp.float32) + b1a[...], 0.0)
    h = jnp.maximum(jnp.dot(h, W1b[...], preferred_element_type=jnp.float32) + b1b[...], 0.0)
    h = jnp.dot(h, W1c[...], preferred_element_type=jnp.float32) + b1c[...]
    h = jnp.maximum(h, 0.0)
    out[...] = _bn(h, g1[...], be1[...])


def _mlp2_body(x, p, W2a, b2a, W2b, b2b, W2c, b2c, g2, be2, out):
    a = x[...] + p[0:N] + p[N_PAD:N_PAD + N]
    h = jnp.maximum(jnp.dot(a, W2a[...], preferred_element_type=jnp.float32) + b2a[...], 0.0)
    h = jnp.maximum(jnp.dot(h, W2b[...], preferred_element_type=jnp.float32) + b2b[...], 0.0)
    h = jnp.dot(h, W2c[...], preferred_element_type=jnp.float32) + b2c[...]
    h = jnp.maximum(h, 0.0)
    out[...] = _bn(h, g2[...], be2[...])


def _mlp3_body(x, p, W3a, b3a, W3b, b3b, g3, be3, Wf1, bf1, Wf2, bf2, out):
    a = x[...] + p[0:N] + p[N_PAD:N_PAD + N]
    h = jnp.maximum(jnp.dot(a, W3a[...], preferred_element_type=jnp.float32) + b3a[...], 0.0)
    h = jnp.dot(h, W3b[...], preferred_element_type=jnp.float32) + b3b[...]
    h = jnp.maximum(h, 0.0)
    h = _bn(h, g3[...], be3[...])
    h = jnp.maximum(jnp.dot(h, Wf1[...], preferred_element_type=jnp.float32) + bf1[...], 0.0)
    h = jnp.dot(h, Wf2[...], preferred_element_type=jnp.float32) + bf2[...]
    out[...] = jnp.tanh(h)


def _tc_call(body, n_out):
    return pl.pallas_call(
        body,
        out_shape=jax.ShapeDtypeStruct((N, n_out), jnp.float32),
    )


def _r(v):
    return v.reshape(1, -1)


def kernel(x, edge_index, batch,
           W1a, b1a, W1b, b1b, W1c, b1c, g1, be1,
           W2a, b2a, W2b, b2b, W2c, b2c, g2, be2,
           W3a, b3a, W3b, b3b, g3, be3,
           Wf1, bf1, Wf2, bf2):
    src = edge_index[0]
    dst = edge_index[1]
    padn = E_PAD - E
    pad_iota = lax.iota(jnp.int32, padn)
    srcp = jnp.concatenate([src, pad_iota % N]).reshape(NW, NCHUNK, K)
    dstp = jnp.concatenate([dst, N + pad_iota % (N_PAD - N)]).reshape(NW, NCHUNK, K)

    p1 = _make_segsum(128)(x, srcp, dstp)
    h1 = _tc_call(_mlp1_body, 128)(
        x, p1, W1a, _r(b1a), W1b, _r(b1b), W1c, _r(b1c), _r(g1), _r(be1))
    p2 = _make_segsum(128)(h1, srcp, dstp)
    h2 = _tc_call(_mlp2_body, 32)(
        h1, p2, W2a, _r(b2a), W2b, _r(b2b), W2c, _r(b2c), _r(g2), _r(be2))
    p3 = _make_segsum(32)(h2, srcp, dstp)
    out = _tc_call(_mlp3_body, 10)(
        h2, p3, W3a, _r(b3a), W3b, _r(b3b), _r(g3), _r(be3),
        Wf1, _r(bf1), Wf2, _r(bf2))
    return out
